# speedup vs baseline: 1.5302x; 1.5302x over previous
import functools

import jax
import jax.numpy as jnp
from jax import lax
from jax.experimental import pallas as pl
from jax.experimental.pallas import tpu as pltpu
from jax.experimental.pallas import tpu_sc as plsc

D_MODEL = 768
NUM_HEADS = 12


@functools.lru_cache(maxsize=None)
def _make_gather(B: int, D: int):
    info = plsc.get_sparse_core_info()
    NC, NS = info.num_cores, info.num_subcores
    NW = NC * NS
    assert B % NW == 0
    b_per_w = B // NW
    C = 128
    n_chunks = b_per_w // C
    mesh = plsc.VectorSubcoreMesh(core_axis_name="c", subcore_axis_name="s")

    @functools.partial(
        pl.kernel,
        mesh=mesh,
        out_type=jax.ShapeDtypeStruct((B, D), jnp.float32),
        scratch_types=[
            pltpu.VMEM((b_per_w,), jnp.int32),
            pltpu.VMEM((C, D), jnp.float32),
            pltpu.SemaphoreType.DMA,
        ],
    )
    def gather_kernel(table_hbm, idx_hbm, out_hbm, idx_v, rows_v, gsem):
        wid = lax.axis_index("s") * NC + lax.axis_index("c")
        base = wid * b_per_w
        pltpu.sync_copy(idx_hbm.at[pl.ds(base, b_per_w)], idx_v)
        for c in range(n_chunks):
            pltpu.async_copy(
                table_hbm.at[idx_v.at[pl.ds(c * C, C)]], rows_v, gsem
            ).wait()
            pltpu.sync_copy(rows_v, out_hbm.at[pl.ds(base + c * C, C)])

    return gather_kernel


def _mask_body(m_ref, o_ref):
    o_ref[...] = (1.0 - m_ref[...]) * -10000.0


def kernel(encoder_input_ids, encoder_attention_mask, embed_table):
    batch, seq = encoder_input_ids.shape
    B = batch * seq
    idx = encoder_input_ids.reshape(B)
    hidden = _make_gather(B, D_MODEL)(embed_table, idx)
    hidden = hidden.reshape(batch, seq, D_MODEL)
    ext_mask = pl.pallas_call(
        _mask_body,
        out_shape=jax.ShapeDtypeStruct((batch, seq), jnp.float32),
    )(encoder_attention_mask)
    ext_mask = ext_mask[:, None, None, :]
    position_bias = jnp.zeros((batch, NUM_HEADS, seq, 1), dtype=jnp.float32)
    return hidden, ext_mask, position_bias


# speedup vs baseline: 1.5788x; 1.0318x over previous
import functools

import jax
import jax.numpy as jnp
from jax import lax
from jax.experimental import pallas as pl
from jax.experimental.pallas import tpu as pltpu
from jax.experimental.pallas import tpu_sc as plsc

D_MODEL = 768
NUM_HEADS = 12


@functools.lru_cache(maxsize=None)
def _make_gather(B: int, D: int):
    info = plsc.get_sparse_core_info()
    NC, NS = info.num_cores, info.num_subcores
    NW = NC * NS
    assert B % NW == 0
    b_per_w = B // NW
    C = 64
    n_chunks = b_per_w // C
    mesh = plsc.VectorSubcoreMesh(core_axis_name="c", subcore_axis_name="s")

    @functools.partial(
        pl.kernel,
        mesh=mesh,
        out_type=jax.ShapeDtypeStruct((B, D), jnp.float32),
        scratch_types=[
            pltpu.VMEM((b_per_w,), jnp.int32),
            pltpu.VMEM((2, C, D), jnp.float32),
            pltpu.SemaphoreType.DMA,
        ],
    )
    def gather_kernel(table_hbm, idx_hbm, out_hbm, idx_v, rows_v, gsem):
        wid = lax.axis_index("s") * NC + lax.axis_index("c")
        base = wid * b_per_w
        pltpu.sync_copy(idx_hbm.at[pl.ds(base, b_per_w)], idx_v)
        copies = [None, None]
        copies[0] = pltpu.async_copy(
            table_hbm.at[idx_v.at[pl.ds(0, C)]], rows_v.at[0], gsem
        )
        for c in range(n_chunks):
            if c + 1 < n_chunks:
                copies[(c + 1) % 2] = pltpu.async_copy(
                    table_hbm.at[idx_v.at[pl.ds((c + 1) * C, C)]],
                    rows_v.at[(c + 1) % 2],
                    gsem,
                )
            copies[c % 2].wait()
            pltpu.sync_copy(rows_v.at[c % 2], out_hbm.at[pl.ds(base + c * C, C)])

    return gather_kernel


def _mask_body(m_ref, o_ref):
    o_ref[...] = (1.0 - m_ref[...]) * -10000.0


def kernel(encoder_input_ids, encoder_attention_mask, embed_table):
    batch, seq = encoder_input_ids.shape
    B = batch * seq
    idx = encoder_input_ids.reshape(B)
    hidden = _make_gather(B, D_MODEL)(embed_table, idx)
    hidden = hidden.reshape(batch, seq, D_MODEL)
    ext_mask = pl.pallas_call(
        _mask_body,
        out_shape=jax.ShapeDtypeStruct((batch, seq), jnp.float32),
    )(encoder_attention_mask)
    ext_mask = ext_mask[:, None, None, :]
    position_bias = jnp.zeros((batch, NUM_HEADS, seq, 1), dtype=jnp.float32)
    return hidden, ext_mask, position_bias
